# baseline (speedup 1.0000x reference)
import functools

import jax
import jax.numpy as jnp
from jax import lax
from jax.experimental import pallas as pl
from jax.experimental.pallas import tpu as pltpu
from jax.experimental.pallas import tpu_sc as plsc

N = 10000
E = 320000
D = 128
G = 128

NC = 2
NS = 16
NW = NC * NS
EW = E // NW
C = 128
NCH = 80
EP = NW * NCH * C
NP = N + 512
IB = 16
NBLK = NCH // IB
NBUF = 2
RPT = 624
TAIL = N - NS * RPT
TAIL0 = NS * RPT


def _sc_agg_body(h_hbm, src_hbm, dst_hbm, out_hbm, sblk, dblk, rows_v,
                 agg_sh, gsem, ssem, isem):
    cid = lax.axis_index("c")
    sid = lax.axis_index("s")
    wid = sid * NC + cid
    row0 = sid * RPT
    pltpu.sync_copy(h_hbm.at[pl.ds(row0, RPT)], agg_sh.at[pl.ds(row0, RPT)])

    @pl.when(sid == NS - 1)
    def _():
        pltpu.sync_copy(h_hbm.at[pl.ds(TAIL0, TAIL)],
                        agg_sh.at[pl.ds(TAIL0, TAIL)])

    plsc.subcore_barrier()

    def idx_descs(blk, p):
        return (pltpu.make_async_copy(src_hbm.at[wid, pl.ds(blk * IB, IB)],
                                      sblk.at[p], isem),
                pltpu.make_async_copy(dst_hbm.at[wid, pl.ds(blk * IB, IB)],
                                      dblk.at[p], isem))

    def gather_desc(p, j, b):
        return pltpu.make_async_copy(h_hbm.at[sblk.at[p, j]], rows_v.at[b],
                                     gsem.at[b])

    def scatter_desc(p, j, b):
        return pltpu.make_async_copy(rows_v.at[b], agg_sh.at[dblk.at[p, j]],
                                     ssem.at[b])

    for dsc in idx_descs(0, 0):
        dsc.start()
    for dsc in idx_descs(0, 0):
        dsc.wait()
    gather_desc(0, 0, 0).start()

    def blk_body(blk, carry):
        p = blk % 2
        for j in range(IB):
            b = j % 2
            gather_desc(p, j, b).wait()
            scatter_desc(p, j, b).start(add=True)
            if j == 0:
                @pl.when(blk > 0)
                def _():
                    scatter_desc(p, j, 1 - b).wait()

                @pl.when(blk + 1 < NBLK)
                def _():
                    for dsc in idx_descs(blk + 1, 1 - p):
                        dsc.start()
            else:
                scatter_desc(p, j, 1 - b).wait()
            if j < IB - 1:
                gather_desc(p, j + 1, 1 - b).start()
            else:
                @pl.when(blk + 1 < NBLK)
                def _():
                    for dsc in idx_descs(blk + 1, 1 - p):
                        dsc.wait()
                    gather_desc(1 - p, 0, 1 - b).start()
        return carry

    lax.fori_loop(0, NBLK, blk_body, 0)
    scatter_desc((NBLK - 1) % 2, IB - 1, 1).wait()
    plsc.subcore_barrier()
    pltpu.sync_copy(agg_sh.at[pl.ds(row0, RPT)],
                    out_hbm.at[cid, pl.ds(row0, RPT)])

    @pl.when(sid == NS - 1)
    def _():
        pltpu.sync_copy(agg_sh.at[pl.ds(TAIL0, TAIL)],
                        out_hbm.at[cid, pl.ds(TAIL0, TAIL)])


def _sc_agg(h, srcp, dstp):
    mesh = plsc.VectorSubcoreMesh(
        core_axis_name="c", subcore_axis_name="s", num_cores=NC, num_subcores=NS)
    k = pl.kernel(
        _sc_agg_body,
        out_type=jax.ShapeDtypeStruct((NC, N, D), jnp.float32),
        mesh=mesh,
        scratch_types=[
            pltpu.VMEM((2, IB, C), jnp.int32),
            pltpu.VMEM((2, IB, C), jnp.int32),
            pltpu.VMEM((NBUF, C, D), jnp.float32),
            pltpu.VMEM_SHARED((NP, D), jnp.float32),
            pltpu.SemaphoreType.DMA((NBUF,)),
            pltpu.SemaphoreType.DMA((NBUF,)),
            pltpu.SemaphoreType.DMA,
        ],
        name="sc_gin_agg",
    )
    return k(h, srcp, dstp)


def _mlp_bn(h, agg, w1, b1, w2, b2, g, b):
    z = (agg[0] + agg[1]) - h
    a1 = jnp.maximum(
        jax.lax.dot(z, w1[...], precision=jax.lax.Precision.DEFAULT,
                    preferred_element_type=jnp.float32) + b1[...], 0.0)
    y = jax.lax.dot(a1, w2[...], precision=jax.lax.Precision.DEFAULT,
                    preferred_element_type=jnp.float32) + b2[...]
    y = jnp.maximum(y, 0.0)
    mu = jnp.mean(y, axis=0, keepdims=True)
    yc = y - mu
    var = jnp.mean(yc * yc, axis=0, keepdims=True)
    return g[...] * yc * jax.lax.rsqrt(var + 1e-5) + b[...]


def _tc_layer_body(h_ref, agg_ref, w1_ref, b1_ref, w2_ref, b2_ref, g_ref,
                   b_ref, out_ref):
    out_ref[...] = _mlp_bn(h_ref[...], agg_ref[...], w1_ref, b1_ref, w2_ref,
                           b2_ref, g_ref, b_ref)


def _tc_layer(h, agg, w1, b1, w2, b2, g, b):
    return pl.pallas_call(
        _tc_layer_body,
        out_shape=jax.ShapeDtypeStruct((N, D), jnp.float32),
    )(h, agg, w1, b1, w2, b2, g, b)


def _tc_final_body(h_ref, agg_ref, w1_ref, b1_ref, w2_ref, b2_ref, g_ref,
                   b_ref, batch_ref, lw_ref, lb_ref, out_ref, hout_ref):
    hn = _mlp_bn(h_ref[...], agg_ref[...], w1_ref, b1_ref, w2_ref, b2_ref,
                 g_ref, b_ref)
    hout_ref[...] = hn
    gids = jax.lax.broadcasted_iota(jnp.int32, (N, G), 1)
    m = (batch_ref[...] == gids).astype(jnp.float32)
    xpool = jax.lax.dot_general(
        m, hn, (((0,), (0,)), ((), ())),
        precision=jax.lax.Precision.DEFAULT,
        preferred_element_type=jnp.float32)
    out_ref[...] = jax.lax.dot(
        xpool, lw_ref[...], precision=jax.lax.Precision.DEFAULT,
        preferred_element_type=jnp.float32) + lb_ref[...]


def _tc_final(h, agg, w1, b1, w2, b2, g, b, batch2d, lw, lb):
    return pl.pallas_call(
        _tc_final_body,
        out_shape=(
            jax.ShapeDtypeStruct((G, 2 * D), jnp.float32),
            jax.ShapeDtypeStruct((N, D), jnp.float32),
        ),
    )(h, agg, w1, b1, w2, b2, g, b, batch2d, lw, lb)


def kernel(x, edge_index, batch, c0_W1, c0_b1, c0_W2, c0_b2, c1_W1, c1_b1,
           c1_W2, c1_b2, c2_W1, c2_b1, c2_W2, c2_b2, bn0_g, bn0_b, bn1_g,
           bn1_b, bn2_g, bn2_b, lin0_W, lin0_b):
    src = edge_index[0].astype(jnp.int32)
    dst = edge_index[1].astype(jnp.int32)
    perm = jnp.argsort(src)
    src = src[perm]
    dst = dst[perm]
    batch2d = batch.astype(jnp.int32).reshape(N, 1)

    def r2(v):
        return v.reshape(1, D)

    padw = NCH * C - EW
    srcp = jnp.concatenate(
        [src.reshape(NW, EW), jnp.zeros((NW, padw), jnp.int32)], axis=1)
    tr = N + (jnp.arange(padw, dtype=jnp.int32) * 8) % 512
    dstp = jnp.concatenate(
        [dst.reshape(NW, EW), jnp.broadcast_to(tr, (NW, padw))], axis=1)
    srcp = srcp.reshape(NW, NCH, C)
    dstp = dstp.reshape(NW, NCH, C)

    agg = _sc_agg(x, srcp, dstp)
    h1 = _tc_layer(x, agg, c0_W1, r2(c0_b1), c0_W2, r2(c0_b2), r2(bn0_g),
                   r2(bn0_b))
    agg = _sc_agg(h1, srcp, dstp)
    h2 = _tc_layer(h1, agg, c1_W1, r2(c1_b1), c1_W2, r2(c1_b2), r2(bn1_g),
                   r2(bn1_b))
    agg = _sc_agg(h2, srcp, dstp)
    out, h3 = _tc_final(h2, agg, c2_W1, r2(c2_b1), c2_W2, r2(c2_b2),
                        r2(bn2_g), r2(bn2_b), batch2d, lin0_W,
                        lin0_b.reshape(1, 2 * D))
    return (out, h3)


# speedup vs baseline: 4.6339x; 4.6339x over previous
import functools

import jax
import jax.numpy as jnp
from jax import lax
from jax.experimental import pallas as pl
from jax.experimental.pallas import tpu as pltpu
from jax.experimental.pallas import tpu_sc as plsc

N = 10000
E = 320000
D = 128
G = 128

NC = 2
NS = 16
NW = NC * NS
EW = E // NW
C = 125
NCH = 80
NP = N
IB = 16
NBLK = NCH // IB
NBUF = 2
RPT = 624
TAIL = N - NS * RPT
TAIL0 = NS * RPT


def _sc_agg_body(h_hbm, src_hbm, dst_hbm, out_hbm, sblk, dblk, rows_v,
                 agg_sh, gsem, ssem, isem):
    cid = lax.axis_index("c")
    sid = lax.axis_index("s")
    wid = sid * NC + cid
    row0 = sid * RPT
    pltpu.sync_copy(h_hbm.at[pl.ds(row0, RPT)], agg_sh.at[pl.ds(row0, RPT)])

    @pl.when(sid == NS - 1)
    def _():
        pltpu.sync_copy(h_hbm.at[pl.ds(TAIL0, TAIL)],
                        agg_sh.at[pl.ds(TAIL0, TAIL)])

    plsc.subcore_barrier()

    def idx_descs(blk, p):
        return (pltpu.make_async_copy(src_hbm.at[wid, pl.ds(blk * IB, IB)],
                                      sblk.at[p], isem),
                pltpu.make_async_copy(dst_hbm.at[wid, pl.ds(blk * IB, IB)],
                                      dblk.at[p], isem))

    def gather_desc(p, j, b):
        return pltpu.make_async_copy(h_hbm.at[sblk.at[p, j]], rows_v.at[b],
                                     gsem.at[b])

    def scatter_desc(p, j, b):
        return pltpu.make_async_copy(rows_v.at[b], agg_sh.at[dblk.at[p, j]],
                                     ssem.at[b])

    for dsc in idx_descs(0, 0):
        dsc.start()
    for dsc in idx_descs(0, 0):
        dsc.wait()
    gather_desc(0, 0, 0).start()

    def blk_body(blk, carry):
        p = blk % 2
        for j in range(IB):
            b = j % 2
            gather_desc(p, j, b).wait()
            scatter_desc(p, j, b).start(add=True)
            if j == 0:
                @pl.when(blk > 0)
                def _():
                    scatter_desc(p, j, 1 - b).wait()

                @pl.when(blk + 1 < NBLK)
                def _():
                    for dsc in idx_descs(blk + 1, 1 - p):
                        dsc.start()
            else:
                scatter_desc(p, j, 1 - b).wait()
            if j < IB - 1:
                gather_desc(p, j + 1, 1 - b).start()
            else:
                @pl.when(blk + 1 < NBLK)
                def _():
                    for dsc in idx_descs(blk + 1, 1 - p):
                        dsc.wait()
                    gather_desc(1 - p, 0, 1 - b).start()
        return carry

    lax.fori_loop(0, NBLK, blk_body, 0)
    scatter_desc((NBLK - 1) % 2, IB - 1, 1).wait()
    plsc.subcore_barrier()
    pltpu.sync_copy(agg_sh.at[pl.ds(row0, RPT)],
                    out_hbm.at[cid, pl.ds(row0, RPT)])

    @pl.when(sid == NS - 1)
    def _():
        pltpu.sync_copy(agg_sh.at[pl.ds(TAIL0, TAIL)],
                        out_hbm.at[cid, pl.ds(TAIL0, TAIL)])


def _sc_agg(h, srcp, dstp):
    mesh = plsc.VectorSubcoreMesh(
        core_axis_name="c", subcore_axis_name="s", num_cores=NC, num_subcores=NS)
    k = pl.kernel(
        _sc_agg_body,
        out_type=jax.ShapeDtypeStruct((NC, N, D), jnp.float32),
        mesh=mesh,
        scratch_types=[
            pltpu.VMEM((2, IB, C), jnp.int32),
            pltpu.VMEM((2, IB, C), jnp.int32),
            pltpu.VMEM((NBUF, C, D), jnp.float32),
            pltpu.VMEM_SHARED((NP, D), jnp.float32),
            pltpu.SemaphoreType.DMA((NBUF,)),
            pltpu.SemaphoreType.DMA((NBUF,)),
            pltpu.SemaphoreType.DMA,
        ],
        name="sc_gin_agg",
    )
    return k(h, srcp, dstp)


def _mlp_bn(h, agg, w1, b1, w2, b2, g, b):
    z = (agg[0] + agg[1]) - h
    a1 = jnp.maximum(
        jax.lax.dot(z, w1[...], precision=jax.lax.Precision.DEFAULT,
                    preferred_element_type=jnp.float32) + b1[...], 0.0)
    y = jax.lax.dot(a1, w2[...], precision=jax.lax.Precision.DEFAULT,
                    preferred_element_type=jnp.float32) + b2[...]
    y = jnp.maximum(y, 0.0)
    mu = jnp.mean(y, axis=0, keepdims=True)
    yc = y - mu
    var = jnp.mean(yc * yc, axis=0, keepdims=True)
    return g[...] * yc * jax.lax.rsqrt(var + 1e-5) + b[...]


def _tc_layer_body(h_ref, agg_ref, w1_ref, b1_ref, w2_ref, b2_ref, g_ref,
                   b_ref, out_ref):
    out_ref[...] = _mlp_bn(h_ref[...], agg_ref[...], w1_ref, b1_ref, w2_ref,
                           b2_ref, g_ref, b_ref)


def _tc_layer(h, agg, w1, b1, w2, b2, g, b):
    return pl.pallas_call(
        _tc_layer_body,
        out_shape=jax.ShapeDtypeStruct((N, D), jnp.float32),
    )(h, agg, w1, b1, w2, b2, g, b)


def _tc_final_body(h_ref, agg_ref, w1_ref, b1_ref, w2_ref, b2_ref, g_ref,
                   b_ref, batch_ref, lw_ref, lb_ref, out_ref, hout_ref):
    hn = _mlp_bn(h_ref[...], agg_ref[...], w1_ref, b1_ref, w2_ref, b2_ref,
                 g_ref, b_ref)
    hout_ref[...] = hn
    gids = jax.lax.broadcasted_iota(jnp.int32, (N, G), 1)
    m = (batch_ref[...] == gids).astype(jnp.float32)
    xpool = jax.lax.dot_general(
        m, hn, (((0,), (0,)), ((), ())),
        precision=jax.lax.Precision.DEFAULT,
        preferred_element_type=jnp.float32)
    out_ref[...] = jax.lax.dot(
        xpool, lw_ref[...], precision=jax.lax.Precision.DEFAULT,
        preferred_element_type=jnp.float32) + lb_ref[...]


def _tc_final(h, agg, w1, b1, w2, b2, g, b, batch2d, lw, lb):
    return pl.pallas_call(
        _tc_final_body,
        out_shape=(
            jax.ShapeDtypeStruct((G, 2 * D), jnp.float32),
            jax.ShapeDtypeStruct((N, D), jnp.float32),
        ),
    )(h, agg, w1, b1, w2, b2, g, b, batch2d, lw, lb)


def kernel(x, edge_index, batch, c0_W1, c0_b1, c0_W2, c0_b2, c1_W1, c1_b1,
           c1_W2, c1_b2, c2_W1, c2_b1, c2_W2, c2_b2, bn0_g, bn0_b, bn1_g,
           bn1_b, bn2_g, bn2_b, lin0_W, lin0_b):
    src = edge_index[0].astype(jnp.int32)
    dst = edge_index[1].astype(jnp.int32)
    batch2d = batch.astype(jnp.int32).reshape(N, 1)

    def r2(v):
        return v.reshape(1, D)

    srcp = src.reshape(NW, NCH, C)
    dstp = dst.reshape(NW, NCH, C)

    agg = _sc_agg(x, srcp, dstp)
    h1 = _tc_layer(x, agg, c0_W1, r2(c0_b1), c0_W2, r2(c0_b2), r2(bn0_g),
                   r2(bn0_b))
    agg = _sc_agg(h1, srcp, dstp)
    h2 = _tc_layer(h1, agg, c1_W1, r2(c1_b1), c1_W2, r2(c1_b2), r2(bn1_g),
                   r2(bn1_b))
    agg = _sc_agg(h2, srcp, dstp)
    out, h3 = _tc_final(h2, agg, c2_W1, r2(c2_b1), c2_W2, r2(c2_b2),
                        r2(bn2_g), r2(bn2_b), batch2d, lin0_W,
                        lin0_b.reshape(1, 2 * D))
    return (out, h3)


# speedup vs baseline: 4.6559x; 1.0047x over previous
import jax
import jax.numpy as jnp
from jax import lax
from jax.experimental import pallas as pl
from jax.experimental.pallas import tpu as pltpu
from jax.experimental.pallas import tpu_sc as plsc

N = 10000
E = 320000
D = 128
G = 128

NC = 2
NS = 16
NW = NC * NS
EW = E // NW
C = 125
NCH = 80
NP = N
IB = 16
NBLK = NCH // IB
NBUF = 2
RPT = 624
TAIL = N - NS * RPT
TAIL0 = NS * RPT


def _sc_agg_body(h_hbm, src_hbm, dst_hbm, out_hbm, sblk, dblk, rows_v,
                 agg_sh, gsem, ssem, isem):
    cid = lax.axis_index("c")
    sid = lax.axis_index("s")
    wid = sid * NC + cid
    row0 = sid * RPT
    pltpu.sync_copy(h_hbm.at[pl.ds(row0, RPT)], agg_sh.at[pl.ds(row0, RPT)])

    @pl.when(sid == NS - 1)
    def _():
        pltpu.sync_copy(h_hbm.at[pl.ds(TAIL0, TAIL)],
                        agg_sh.at[pl.ds(TAIL0, TAIL)])

    plsc.subcore_barrier()

    def idx_descs(blk, p):
        return (pltpu.make_async_copy(src_hbm.at[wid, pl.ds(blk * IB, IB)],
                                      sblk.at[p], isem),
                pltpu.make_async_copy(dst_hbm.at[wid, pl.ds(blk * IB, IB)],
                                      dblk.at[p], isem))

    def gather_desc(p, j, b):
        return pltpu.make_async_copy(h_hbm.at[sblk.at[p, j]], rows_v.at[b],
                                     gsem.at[b])

    def scatter_desc(p, j, b):
        return pltpu.make_async_copy(rows_v.at[b], agg_sh.at[dblk.at[p, j]],
                                     ssem.at[b])

    for dsc in idx_descs(0, 0):
        dsc.start()
    for dsc in idx_descs(0, 0):
        dsc.wait()
    gather_desc(0, 0, 0).start()

    def blk_body(blk, carry):
        p = blk % 2
        for j in range(IB):
            b = j % 2
            gather_desc(p, j, b).wait()
            scatter_desc(p, j, b).start(add=True)
            if j == 0:
                @pl.when(blk > 0)
                def _():
                    scatter_desc(p, j, 1 - b).wait()

                @pl.when(blk + 1 < NBLK)
                def _():
                    for dsc in idx_descs(blk + 1, 1 - p):
                        dsc.start()
            else:
                scatter_desc(p, j, 1 - b).wait()
            if j < IB - 1:
                gather_desc(p, j + 1, 1 - b).start()
            else:
                @pl.when(blk + 1 < NBLK)
                def _():
                    for dsc in idx_descs(blk + 1, 1 - p):
                        dsc.wait()
                    gather_desc(1 - p, 0, 1 - b).start()
        return carry

    lax.fori_loop(0, NBLK, blk_body, 0)
    scatter_desc((NBLK - 1) % 2, IB - 1, 1).wait()
    plsc.subcore_barrier()
    pltpu.sync_copy(agg_sh.at[pl.ds(row0, RPT)],
                    out_hbm.at[cid, pl.ds(row0, RPT)])

    @pl.when(sid == NS - 1)
    def _():
        pltpu.sync_copy(agg_sh.at[pl.ds(TAIL0, TAIL)],
                        out_hbm.at[cid, pl.ds(TAIL0, TAIL)])


def _sc_agg(h, srcp, dstp):
    mesh = plsc.VectorSubcoreMesh(
        core_axis_name="c", subcore_axis_name="s", num_cores=NC, num_subcores=NS)
    k = pl.kernel(
        _sc_agg_body,
        out_type=jax.ShapeDtypeStruct((NC, N, D), jnp.float32),
        mesh=mesh,
        scratch_types=[
            pltpu.VMEM((2, IB, C), jnp.int32),
            pltpu.VMEM((2, IB, C), jnp.int32),
            pltpu.VMEM((NBUF, C, D), jnp.float32),
            pltpu.VMEM_SHARED((NP, D), jnp.float32),
            pltpu.SemaphoreType.DMA((NBUF,)),
            pltpu.SemaphoreType.DMA((NBUF,)),
            pltpu.SemaphoreType.DMA,
        ],
        name="sc_gin_agg",
    )
    return k(h, srcp, dstp)


def _mlp_bn(h, agg, w1, b1, w2, b2, g, b):
    z = (agg[0] + agg[1]) - h
    a1 = jnp.maximum(
        jax.lax.dot(z, w1[...], precision=jax.lax.Precision.DEFAULT,
                    preferred_element_type=jnp.float32) + b1[...], 0.0)
    y = jax.lax.dot(a1, w2[...], precision=jax.lax.Precision.DEFAULT,
                    preferred_element_type=jnp.float32) + b2[...]
    y = jnp.maximum(y, 0.0)
    mu = jnp.mean(y, axis=0, keepdims=True)
    yc = y - mu
    var = jnp.mean(yc * yc, axis=0, keepdims=True)
    return g[...] * yc * jax.lax.rsqrt(var + 1e-5) + b[...]


def _tc_layer_body(h_ref, agg_ref, w1_ref, b1_ref, w2_ref, b2_ref, g_ref,
                   b_ref, out_ref):
    out_ref[...] = _mlp_bn(h_ref[...], agg_ref[...], w1_ref, b1_ref, w2_ref,
                           b2_ref, g_ref, b_ref)


def _tc_layer(h, agg, w1, b1, w2, b2, g, b):
    return pl.pallas_call(
        _tc_layer_body,
        out_shape=jax.ShapeDtypeStruct((N, D), jnp.float32),
    )(h, agg, w1, b1, w2, b2, g, b)


def _tc_final_body(h_ref, agg_ref, w1_ref, b1_ref, w2_ref, b2_ref, g_ref,
                   b_ref, batch_ref, lw_ref, lb_ref, out_ref, hout_ref):
    hn = _mlp_bn(h_ref[...], agg_ref[...], w1_ref, b1_ref, w2_ref, b2_ref,
                 g_ref, b_ref)
    hout_ref[...] = hn
    gids = jax.lax.broadcasted_iota(jnp.int32, (N, G), 1)
    m = (batch_ref[...] == gids).astype(jnp.float32)
    xpool = jax.lax.dot_general(
        m, hn, (((0,), (0,)), ((), ())),
        precision=jax.lax.Precision.DEFAULT,
        preferred_element_type=jnp.float32)
    out_ref[...] = jax.lax.dot(
        xpool, lw_ref[...], precision=jax.lax.Precision.DEFAULT,
        preferred_element_type=jnp.float32) + lb_ref[...]


def _tc_final(h, agg, w1, b1, w2, b2, g, b, batch2d, lw, lb):
    return pl.pallas_call(
        _tc_final_body,
        out_shape=(
            jax.ShapeDtypeStruct((G, 2 * D), jnp.float32),
            jax.ShapeDtypeStruct((N, D), jnp.float32),
        ),
    )(h, agg, w1, b1, w2, b2, g, b, batch2d, lw, lb)


def kernel(x, edge_index, batch, c0_W1, c0_b1, c0_W2, c0_b2, c1_W1, c1_b1,
           c1_W2, c1_b2, c2_W1, c2_b1, c2_W2, c2_b2, bn0_g, bn0_b, bn1_g,
           bn1_b, bn2_g, bn2_b, lin0_W, lin0_b):
    src = edge_index[0].astype(jnp.int32)
    dst = edge_index[1].astype(jnp.int32)
    batch2d = batch.astype(jnp.int32).reshape(N, 1)

    def r2(v):
        return v.reshape(1, D)

    srcp = src.reshape(NW, NCH, C)
    dstp = dst.reshape(NW, NCH, C)

    agg = _sc_agg(x, srcp, dstp)
    h1 = _tc_layer(x, agg, c0_W1, r2(c0_b1), c0_W2, r2(c0_b2), r2(bn0_g),
                   r2(bn0_b))
    agg = _sc_agg(h1, srcp, dstp)
    h2 = _tc_layer(h1, agg, c1_W1, r2(c1_b1), c1_W2, r2(c1_b2), r2(bn1_g),
                   r2(bn1_b))
    agg = _sc_agg(h2, srcp, dstp)
    out, h3 = _tc_final(h2, agg, c2_W1, r2(c2_b1), c2_W2, r2(c2_b2),
                        r2(bn2_g), r2(bn2_b), batch2d, lin0_W,
                        lin0_b.reshape(1, 2 * D))
    return (out, h3)
